# baseline (speedup 1.0000x reference)
import functools

import jax
import jax.numpy as jnp
from jax import lax
from jax.experimental import pallas as pl
from jax.experimental.pallas import tpu as pltpu
from jax.experimental.pallas import tpu_sc as plsc

EMBED = 64
LANES = 16
NC, NS = 2, 16
NW = NC * NS
BATCH = 16
CTX = 2048
TOTAL = BATCH * CTX
BPW = TOTAL // NW
CHUNK = 128
NCHUNK = BPW // CHUNK
SCALE = 8.0
VOCAB2 = 500000


def _sc_embed(x2d, tab2, pos128):
    mesh = plsc.VectorSubcoreMesh(core_axis_name="c", subcore_axis_name="s")

    @functools.partial(
        pl.kernel,
        out_type=jax.ShapeDtypeStruct((TOTAL // 2, 2 * EMBED), jnp.float32),
        mesh=mesh,
        scratch_types=[
            pltpu.VMEM((NCHUNK, CHUNK), jnp.int32),
            pltpu.VMEM((NCHUNK, CHUNK), jnp.int32),
            pltpu.VMEM((NCHUNK, CHUNK), jnp.int32),
            pltpu.VMEM((CHUNK, 2 * EMBED), jnp.float32),
            pltpu.VMEM((CHUNK // 2, 2 * EMBED), jnp.float32),
            pltpu.VMEM((CHUNK // 2, 2 * EMBED), jnp.float32),
            pltpu.SemaphoreType.DMA,
        ],
        compiler_params=pltpu.CompilerParams(needs_layout_passes=False),
    )
    def k(x_hbm, tab_hbm, pos_hbm, out_hbm, idx_v, q_v, cb_v, pairs_v,
          out_v, pos_v, sem):
        wid = lax.axis_index("s") * NC + lax.axis_index("c")
        obase = wid * (BPW // 2)
        pbase = (wid % 2) * (BPW // 2)

        pltpu.sync_copy(x_hbm.at[pl.ds(wid * NCHUNK, NCHUNK)], idx_v)

        def prep_body(rr, _):
            for cc in range(CHUNK // LANES):
                sl = (rr, pl.ds(cc * LANES, LANES))
                v = idx_v[sl]
                q_v[sl] = lax.shift_right_arithmetic(v, 1)
                cb_v[sl] = lax.shift_left(jnp.bitwise_and(v, 1), 6)
            return 0

        lax.fori_loop(0, NCHUNK, prep_body, 0)

        lane = lax.iota(jnp.int32, LANES)

        for j in range(NCHUNK):
            pltpu.async_copy(tab_hbm.at[q_v.at[j]], pairs_v, sem).wait()
            pltpu.sync_copy(
                pos_hbm.at[pl.ds(pbase + j * (CHUNK // 2), CHUNK // 2)], pos_v)

            jfull = jnp.full((LANES,), j, dtype=jnp.int32)

            def row_body(r, _):
                for half in range(2):
                    jj = 2 * r + half
                    jjv = jnp.full((LANES,), jj, dtype=jnp.int32)
                    cb = plsc.load_gather(cb_v, [jfull, jjv])
                    for c in range(EMBED // LANES):
                        cols = cb + (c * LANES + lane)
                        src = plsc.load_gather(pairs_v, [jjv, cols])
                        dst = (r, pl.ds(half * EMBED + c * LANES, LANES))
                        out_v[dst] = src * SCALE + pos_v[dst]
                return 0

            lax.fori_loop(0, CHUNK // 2, row_body, 0)
            pltpu.sync_copy(
                out_v, out_hbm.at[pl.ds(obase + j * (CHUNK // 2), CHUNK // 2)])

    return k(x2d, tab2, pos128)


def kernel(X, tok_table, pos_embedding):
    x2d = X.reshape(TOTAL // CHUNK, CHUNK)
    tab2 = tok_table.reshape(VOCAB2, 2 * EMBED)
    pos128 = pos_embedding.reshape(CTX // 2, 2 * EMBED)
    out = _sc_embed(x2d, tab2, pos128)
    return out.reshape(BATCH, CTX, EMBED)


# speedup vs baseline: 1.5564x; 1.5564x over previous
import functools

import jax
import jax.numpy as jnp
from jax import lax
from jax.experimental import pallas as pl
from jax.experimental.pallas import tpu as pltpu
from jax.experimental.pallas import tpu_sc as plsc

EMBED = 64
LANES = 16
NC, NS = 2, 16
NW = NC * NS
BATCH = 16
CTX = 2048
TOTAL = BATCH * CTX
BPW = TOTAL // NW
CHUNK = 128
NCHUNK = BPW // CHUNK
K = 16
SCALE = 8.0


def _sc_embed(X, tab, pos):
    mesh = plsc.VectorSubcoreMesh(core_axis_name="c", subcore_axis_name="s")

    @functools.partial(
        pl.kernel,
        out_type=jax.ShapeDtypeStruct((BATCH, CTX, EMBED), jnp.float32),
        mesh=mesh,
        scratch_types=[
            pltpu.VMEM((BPW,), jnp.int32),
            pltpu.VMEM((CHUNK, EMBED), jnp.float32),
            pltpu.VMEM((CHUNK, EMBED), jnp.float32),
            pltpu.SemaphoreType.DMA,
        ],
        compiler_params=pltpu.CompilerParams(
            needs_layout_passes=False, use_tc_tiling_on_sc=True),
    )
    def k(x_hbm, tab_hbm, pos_hbm, out_hbm, idx_v, rows_v, pos_v, sem):
        wid = lax.axis_index("s") * NC + lax.axis_index("c")
        b = wid // 2
        t0 = (wid % 2) * BPW

        pltpu.sync_copy(x_hbm.at[b, pl.ds(t0, BPW)], idx_v)

        for j in range(NCHUNK):
            for g in range(CHUNK // K):
                v16 = idx_v[pl.ds(j * CHUNK + g * K, LANES)]
                handles = []
                for r in range(K):
                    rr = g * K + r
                    handles.append(pltpu.async_copy(
                        tab_hbm.at[pl.ds(v16[r], 1)],
                        rows_v.at[pl.ds(rr, 1)], sem))
                for h in handles:
                    h.wait()
            pltpu.sync_copy(
                pos_hbm.at[0, pl.ds(t0 + j * CHUNK, CHUNK)], pos_v)

            def row_body(r, _):
                for c in range(EMBED // LANES):
                    sl = (r, pl.ds(c * LANES, LANES))
                    rows_v[sl] = rows_v[sl] * SCALE + pos_v[sl]
                return 0

            lax.fori_loop(0, CHUNK, row_body, 0)
            pltpu.sync_copy(
                rows_v, out_hbm.at[b, pl.ds(t0 + j * CHUNK, CHUNK)])

    return k(X, tab, pos)


def kernel(X, tok_table, pos_embedding):
    return _sc_embed(X, tok_table, pos_embedding)


# speedup vs baseline: 1.7515x; 1.1253x over previous
import functools

import jax
import jax.numpy as jnp
from jax import lax
from jax.experimental import pallas as pl
from jax.experimental.pallas import tpu as pltpu
from jax.experimental.pallas import tpu_sc as plsc

EMBED = 64
LANES = 16
NC, NS = 2, 16
NW = NC * NS
BATCH = 16
CTX = 2048
TOTAL = BATCH * CTX
BPW = TOTAL // NW
CHUNK = 128
NCHUNK = BPW // CHUNK
SCALE = 8.0


def _sc_embed(X, tab, pos):
    mesh = plsc.VectorSubcoreMesh(core_axis_name="c", subcore_axis_name="s")

    @functools.partial(
        pl.kernel,
        out_type=jax.ShapeDtypeStruct((BATCH, CTX, EMBED), jnp.float32),
        mesh=mesh,
        scratch_types=[
            pltpu.VMEM((BPW,), jnp.int32),
            pltpu.VMEM((CHUNK, EMBED), jnp.float32),
            pltpu.VMEM((CHUNK, EMBED), jnp.float32),
            pltpu.VMEM((CHUNK, EMBED), jnp.float32),
            pltpu.VMEM((CHUNK, EMBED), jnp.float32),
            pltpu.SemaphoreType.DMA,
            pltpu.SemaphoreType.DMA,
            pltpu.SemaphoreType.DMA,
            pltpu.SemaphoreType.DMA,
            pltpu.SemaphoreType.DMA,
            pltpu.SemaphoreType.DMA,
        ],
        compiler_params=pltpu.CompilerParams(
            needs_layout_passes=False, use_tc_tiling_on_sc=True),
    )
    def k(x_hbm, tab_hbm, pos_hbm, out_hbm, idx_v, buf_a, buf_b,
          pos_a, pos_b, gsem_a, gsem_b, wsem_a, wsem_b, psem_a, psem_b):
        wid = lax.axis_index("s") * NC + lax.axis_index("c")
        b = wid // 2
        t0 = (wid % 2) * BPW

        bufs = (buf_a, buf_b)
        poss = (pos_a, pos_b)
        gsems = (gsem_a, gsem_b)
        wsems = (wsem_a, wsem_b)
        psems = (psem_a, psem_b)

        pltpu.sync_copy(x_hbm.at[b, pl.ds(t0, BPW)], idx_v)

        def fire_chunk(j, par):
            buf, gsem = bufs[par], gsems[par]
            pltpu.async_copy(
                pos_hbm.at[0, pl.ds(t0 + j * CHUNK, CHUNK)],
                poss[par], psems[par])

            def fire_group(i, _):
                v16 = idx_v[pl.ds(j * CHUNK + i * LANES, LANES)]
                for r in range(LANES):
                    pltpu.async_copy(
                        tab_hbm.at[pl.ds(v16[r], 1)],
                        buf.at[pl.ds(i * LANES + r, 1)], gsem)
                return 0

            lax.fori_loop(0, CHUNK // LANES, fire_group, 0)

        def drain_chunk(par):
            pltpu.make_async_copy(
                tab_hbm.at[pl.ds(0, CHUNK)], bufs[par], gsems[par]).wait()
            pltpu.make_async_copy(
                pos_hbm.at[0, pl.ds(0, CHUNK)], poss[par], psems[par]).wait()

        fire_chunk(0, 0)

        wb = [None] * NCHUNK
        for j in range(NCHUNK):
            par = j % 2
            if j + 1 < NCHUNK:
                if j >= 1 and wb[j - 1] is not None:
                    wb[j - 1].wait()
                    wb[j - 1] = None
                fire_chunk(j + 1, 1 - par)
            drain_chunk(par)

            buf, posb = bufs[par], poss[par]

            def row_body(r, _):
                for c in range(EMBED // LANES):
                    sl = (r, pl.ds(c * LANES, LANES))
                    buf[sl] = buf[sl] * SCALE + posb[sl]
                return 0

            lax.fori_loop(0, CHUNK, row_body, 0)
            wb[j] = pltpu.async_copy(
                buf, out_hbm.at[b, pl.ds(t0 + j * CHUNK, CHUNK)], wsems[par])

        for h in wb:
            if h is not None:
                h.wait()

    return k(X, tab, pos)


def kernel(X, tok_table, pos_embedding):
    return _sc_embed(X, tok_table, pos_embedding)


# speedup vs baseline: 1.7610x; 1.0054x over previous
import functools

import jax
import jax.numpy as jnp
from jax import lax
from jax.experimental import pallas as pl
from jax.experimental.pallas import tpu as pltpu
from jax.experimental.pallas import tpu_sc as plsc

EMBED = 64
LANES = 16
NC, NS = 2, 16
NW = NC * NS
BATCH = 16
CTX = 2048
TOTAL = BATCH * CTX
BPW = TOTAL // NW
CHUNK = 128
NCHUNK = BPW // CHUNK
SCALE = 8.0


def _sc_embed(X, tab, pos):
    mesh = plsc.VectorSubcoreMesh(core_axis_name="c", subcore_axis_name="s")

    @functools.partial(
        pl.kernel,
        out_type=jax.ShapeDtypeStruct((BATCH, CTX, EMBED), jnp.float32),
        mesh=mesh,
        scratch_types=[
            pltpu.VMEM((BPW,), jnp.int32),
            pltpu.VMEM((CHUNK, EMBED), jnp.float32),
            pltpu.VMEM((CHUNK, EMBED), jnp.float32),
            pltpu.VMEM((CHUNK, EMBED), jnp.float32),
            pltpu.VMEM((CHUNK, EMBED), jnp.float32),
            pltpu.SemaphoreType.DMA,
            pltpu.SemaphoreType.DMA,
            pltpu.SemaphoreType.DMA,
            pltpu.SemaphoreType.DMA,
            pltpu.SemaphoreType.DMA,
            pltpu.SemaphoreType.DMA,
            pltpu.SemaphoreType.DMA,
            pltpu.SemaphoreType.DMA,
            pltpu.SemaphoreType.DMA,
            pltpu.SemaphoreType.DMA,
            pltpu.SemaphoreType.DMA,
            pltpu.SemaphoreType.DMA,
        ],
        compiler_params=pltpu.CompilerParams(
            needs_layout_passes=False, use_tc_tiling_on_sc=True),
    )
    def k(x_hbm, tab_hbm, pos_hbm, out_hbm, idx_v, buf_a, buf_b,
          pos_a, pos_b, ga0, ga1, ga2, ga3, gb0, gb1, gb2, gb3,
          wsem_a, wsem_b, psem_a, psem_b):
        wid = lax.axis_index("s") * NC + lax.axis_index("c")
        b = wid // 2
        t0 = (wid % 2) * BPW

        bufs = (buf_a, buf_b)
        poss = (pos_a, pos_b)
        gsems = ((ga0, ga1, ga2, ga3), (gb0, gb1, gb2, gb3))
        wsems = (wsem_a, wsem_b)
        psems = (psem_a, psem_b)

        pltpu.sync_copy(x_hbm.at[b, pl.ds(t0, BPW)], idx_v)

        def fire_chunk(j, par):
            buf, gsem = bufs[par], gsems[par]
            pltpu.async_copy(
                pos_hbm.at[0, pl.ds(t0 + j * CHUNK, CHUNK)],
                poss[par], psems[par])

            def fire_group(i, _):
                v16 = idx_v[pl.ds(j * CHUNK + i * LANES, LANES)]
                for r in range(LANES):
                    pltpu.async_copy(
                        tab_hbm.at[pl.ds(v16[r], 1)],
                        buf.at[pl.ds(i * LANES + r, 1)], gsem[r % 4])
                return 0

            lax.fori_loop(0, CHUNK // LANES, fire_group, 0)

        def drain_chunk(par):
            for q in range(4):
                pltpu.make_async_copy(
                    tab_hbm.at[pl.ds(0, CHUNK // 4)],
                    bufs[par].at[pl.ds(0, CHUNK // 4)],
                    gsems[par][q]).wait()
            pltpu.make_async_copy(
                pos_hbm.at[0, pl.ds(0, CHUNK)], poss[par], psems[par]).wait()

        fire_chunk(0, 0)

        wb = [None] * NCHUNK
        for j in range(NCHUNK):
            par = j % 2
            if j + 1 < NCHUNK:
                if j >= 1 and wb[j - 1] is not None:
                    wb[j - 1].wait()
                    wb[j - 1] = None
                fire_chunk(j + 1, 1 - par)
            drain_chunk(par)

            buf, posb = bufs[par], poss[par]

            def row_body(r, _):
                for c in range(EMBED // LANES):
                    sl = (r, pl.ds(c * LANES, LANES))
                    buf[sl] = buf[sl] * SCALE + posb[sl]
                return 0

            lax.fori_loop(0, CHUNK, row_body, 0)
            wb[j] = pltpu.async_copy(
                buf, out_hbm.at[b, pl.ds(t0 + j * CHUNK, CHUNK)], wsems[par])

        for h in wb:
            if h is not None:
                h.wait()

    return k(X, tab, pos)


def kernel(X, tok_table, pos_embedding):
    return _sc_embed(X, tok_table, pos_embedding)
